# baseline (speedup 1.0000x reference)
import functools

import jax
import jax.numpy as jnp
from jax import lax
from jax.experimental import pallas as pl
from jax.experimental.pallas import tpu as pltpu
from jax.experimental.pallas import tpu_sc as plsc

VOCAB = 1000000
D = 64
BATCH = 4096
HIST = 200
NW = 32
BW = BATCH // NW
NBUF = 4

_mesh = plsc.VectorSubcoreMesh(core_axis_name="c", subcore_axis_name="s")

_scratch = (
    [pltpu.VMEM((HIST, BW), jnp.int32)]
    + [pltpu.VMEM((BW, D), jnp.float32) for _ in range(NBUF)]
    + [pltpu.VMEM((8, 8, 128), jnp.float32) for _ in range(NBUF)]
    + [pltpu.SemaphoreType.DMA for _ in range(2 * NBUF)]
)


@functools.partial(
    pl.kernel,
    mesh=_mesh,
    out_type=jax.ShapeDtypeStruct((HIST, 8, NW, 8, 128), jnp.float32),
    scratch_types=_scratch,
    compiler_params=pltpu.CompilerParams(
        use_tc_tiling_on_sc=False, needs_layout_passes=False),
)
def _embed_gather(idx_hbm, table_hbm, out_hbm, idx_all, *bufs):
    rows = list(bufs[0:NBUF])
    trs = list(bufs[NBUF:2 * NBUF])
    sem_g = list(bufs[2 * NBUF:3 * NBUF])
    sem_s = list(bufs[3 * NBUF:4 * NBUF])
    wid = lax.axis_index("s") * 2 + lax.axis_index("c")
    b0 = wid * BW

    pltpu.sync_copy(idx_hbm.at[:, pl.ds(b0, BW)], idx_all)

    lane = lax.iota(jnp.int32, 16)
    rowv = [lane + 16 * bg for bg in range(8)]

    def fire_gather(h, i):
        pltpu.async_copy(table_hbm.at[idx_all.at[h]], rows[i], sem_g[i])

    def wait_gather(h, i):
        pltpu.make_async_copy(
            table_hbm.at[idx_all.at[h]], rows[i], sem_g[i]).wait()

    def transpose2(rA, trA, rB, trB):
        @plsc.parallel_loop(0, D)
        def _(d):
            di = d // 8
            dj = d % 8
            cold = jnp.full((16,), d, jnp.int32)
            for bg in range(8):
                vA = plsc.load_gather(rA, [rowv[bg], cold])
                vB = plsc.load_gather(rB, [rowv[bg], cold])
                trA[di, dj, pl.ds(16 * bg, 16)] = vA
                trB[di, dj, pl.ds(16 * bg, 16)] = vB

    def fire_store(h, i):
        pltpu.async_copy(trs[i], out_hbm.at[h, :, wid], sem_s[i])

    def wait_store(h, i):
        pltpu.make_async_copy(trs[i], out_hbm.at[h, :, wid], sem_s[i]).wait()

    def step2(h, i, first, last):
        wait_gather(h, i)
        wait_gather(h + 1, i + 1)
        if not first:
            wait_store(h - NBUF, i)
            wait_store(h + 1 - NBUF, i + 1)
        transpose2(rows[i], trs[i], rows[i + 1], trs[i + 1])
        if not last:
            fire_gather(h + NBUF, i)
            fire_gather(h + 1 + NBUF, i + 1)
        fire_store(h, i)
        fire_store(h + 1, i + 1)

    for i in range(NBUF):
        fire_gather(i, i)

    def quad(q, first, last):
        h0 = NBUF * q
        step2(h0, 0, first, last)
        step2(h0 + 2, 2, first, last)
        if last:
            for i in range(NBUF):
                wait_store(h0 + i, i)

    quad(0, True, False)

    def body(q, carry):
        quad(q, False, False)
        return carry

    lax.fori_loop(1, HIST // NBUF - 1, body, 0)
    quad(HIST // NBUF - 1, False, True)


def kernel(idx_texts, table):
    idx_t = jnp.transpose(idx_texts).astype(jnp.int32)
    out5 = _embed_gather(idx_t, table)
    return jnp.transpose(out5, (2, 4, 0, 1, 3)).reshape(BATCH, HIST, D)


# speedup vs baseline: 1.1249x; 1.1249x over previous
import functools

import jax
import jax.numpy as jnp
from jax import lax
from jax.experimental import pallas as pl
from jax.experimental.pallas import tpu as pltpu
from jax.experimental.pallas import tpu_sc as plsc

VOCAB = 1000000
D = 64
B_TOTAL = 4096 * 200
NW = 32
B_PER_W = B_TOTAL // NW
IDX_W = 128
IDX_ROWS_PER_W = B_PER_W // IDX_W
G = 5
CHUNK = G * IDX_W
N_CHUNKS = B_PER_W // CHUNK
M_PAIRS = N_CHUNKS // 2

_mesh = plsc.VectorSubcoreMesh(core_axis_name="c", subcore_axis_name="s")


@functools.partial(
    pl.kernel,
    mesh=_mesh,
    out_type=jax.ShapeDtypeStruct((B_TOTAL, D), jnp.float32),
    scratch_types=[
        pltpu.VMEM((IDX_ROWS_PER_W, IDX_W), jnp.int32),
        pltpu.VMEM((CHUNK, D), jnp.float32),
        pltpu.VMEM((CHUNK, D), jnp.float32),
        pltpu.SemaphoreType.DMA,
        pltpu.SemaphoreType.DMA,
        pltpu.SemaphoreType.DMA,
        pltpu.SemaphoreType.DMA,
    ],
    compiler_params=pltpu.CompilerParams(use_tc_tiling_on_sc=False),
)
def _embed_gather(idx_hbm, table_hbm, out_hbm, idx_all, rows0, rows1,
                  sem_g0, sem_g1, sem_s0, sem_s1):
    wid = lax.axis_index("s") * 2 + lax.axis_index("c")
    idx_row0 = wid * IDX_ROWS_PER_W
    out_row0 = wid * B_PER_W

    pltpu.sync_copy(idx_hbm.at[pl.ds(idx_row0, IDX_ROWS_PER_W)], idx_all)

    def fire_gathers(j, rows, sem):
        r0 = j * G
        return [
            pltpu.async_copy(
                table_hbm.at[idx_all.at[r0 + g]],
                rows.at[pl.ds(g * IDX_W, IDX_W)],
                sem,
            )
            for g in range(G)
        ]

    def fire_store(j, rows, sem):
        return pltpu.async_copy(
            rows, out_hbm.at[pl.ds(out_row0 + j * CHUNK, CHUNK)], sem)

    def wait_store(j, rows, sem):
        pltpu.make_async_copy(
            rows, out_hbm.at[pl.ds(out_row0 + j * CHUNK, CHUNK)], sem).wait()

    def pair(m, first, last):
        j0 = 2 * m
        j1 = j0 + 1
        if not first:
            wait_store(j0 - 2, rows0, sem_s0)
        g0 = fire_gathers(j0, rows0, sem_g0)
        if not first:
            wait_store(j1 - 2, rows1, sem_s1)
        g1 = fire_gathers(j1, rows1, sem_g1)
        for c in g0:
            c.wait()
        fire_store(j0, rows0, sem_s0)
        for c in g1:
            c.wait()
        fire_store(j1, rows1, sem_s1)
        if last:
            wait_store(j0, rows0, sem_s0)
            wait_store(j1, rows1, sem_s1)

    pair(0, True, False)

    def body(m, carry):
        pair(m, False, False)
        return carry

    lax.fori_loop(1, M_PAIRS - 1, body, 0)
    pair(M_PAIRS - 1, False, True)


def kernel(idx_texts, table):
    idx_flat = idx_texts.reshape(B_TOTAL // IDX_W, IDX_W).astype(jnp.int32)
    out = _embed_gather(idx_flat, table)
    return out.reshape(idx_texts.shape + (D,))
